# baseline (speedup 1.0000x reference)
import functools

import jax
import jax.numpy as jnp
from jax.experimental import pallas as pl
from jax.experimental.pallas import tpu as pltpu

_TB = 128
_RPB = 2


def _dan_kernel(ids_smem, e8_ref, w1_ref, b1_ref, w2_ref, b2_ref,
                out_ref, mean_ref, *, seq_len, tb):
    gi = pl.program_id(0)
    d = e8_ref.shape[2]
    iota8 = jax.lax.broadcasted_iota(jnp.int32, (8, d), 0)

    def body(g, carry):
        row0 = g * _RPB
        base = (gi * tb + row0) * seq_len
        for r in range(_RPB):
            rowbase = base + r * seq_len
            idx = ids_smem[rowbase]
            acc = jnp.where(iota8 == (idx & 7), e8_ref[idx >> 3], 0.0)
            for s in range(1, seq_len):
                idx = ids_smem[rowbase + s]
                acc = acc + jnp.where(iota8 == (idx & 7),
                                      e8_ref[idx >> 3], 0.0)
            mean_ref[row0 + r, 0, :] = jnp.sum(acc, axis=0)
        return carry

    jax.lax.fori_loop(0, tb // _RPB, body, 0)

    mean = mean_ref[...].reshape(tb, d) * (1.0 / seq_len)
    h = jnp.dot(mean, w1_ref[...],
                preferred_element_type=jnp.float32) + b1_ref[...]
    h = jnp.maximum(h, 0.0)
    logits = jnp.dot(h, w2_ref[...],
                     preferred_element_type=jnp.float32) + b2_ref[...]
    m = jnp.max(logits, axis=1, keepdims=True)
    shifted = logits - m
    lse = jnp.log(jnp.sum(jnp.exp(shifted), axis=1, keepdims=True))
    out_ref[...] = shifted - lse


def kernel(ids, emb, w1, b1, w2, b2):
    B, S = ids.shape
    V, D = emb.shape
    H = w1.shape[1]
    O = w2.shape[1]

    nb = pl.cdiv(B, _TB)
    Bp = nb * _TB
    ids_p = ids
    if Bp != B:
        ids_p = jnp.zeros((Bp, S), jnp.int32).at[:B, :].set(ids)
    ids_flat = ids_p.reshape(Bp * S)

    e8 = emb.reshape(V // 8, 8, D)

    out = pl.pallas_call(
        functools.partial(_dan_kernel, seq_len=S, tb=_TB),
        out_shape=jax.ShapeDtypeStruct((Bp, O), jnp.float32),
        grid=(nb,),
        in_specs=[
            pl.BlockSpec(memory_space=pltpu.SMEM),
            pl.BlockSpec((V // 8, 8, D), lambda i: (0, 0, 0)),
            pl.BlockSpec((D, H), lambda i: (0, 0)),
            pl.BlockSpec((1, H), lambda i: (0, 0)),
            pl.BlockSpec((H, O), lambda i: (0, 0)),
            pl.BlockSpec((1, O), lambda i: (0, 0)),
        ],
        out_specs=pl.BlockSpec((_TB, O), lambda i: (i, 0)),
        scratch_shapes=[pltpu.VMEM((_TB, 1, D), jnp.float32)],
        compiler_params=pltpu.CompilerParams(
            dimension_semantics=("arbitrary",)),
    )(ids_flat, e8, w1, b1, w2, b2)

    return out[:B, :]


# speedup vs baseline: 1.1156x; 1.1156x over previous
import functools

import jax
import jax.numpy as jnp
from jax.experimental import pallas as pl
from jax.experimental.pallas import tpu as pltpu

_TB = 128
_RPB = 4


def _dan_kernel(ids_smem, e3_ref, w1_ref, b1_ref, w2_ref, b2_ref,
                out_ref, mean_ref, *, seq_len, tb):
    gi = pl.program_id(0)

    def body(g, carry):
        row0 = g * _RPB
        base = (gi * tb + row0) * seq_len
        for r in range(_RPB):
            rowbase = base + r * seq_len
            acc = e3_ref[pl.ds(ids_smem[rowbase], 1), 0, :]
            for s in range(1, seq_len):
                acc = acc + e3_ref[pl.ds(ids_smem[rowbase + s], 1), 0, :]
            mean_ref[row0 + r, 0, :] = acc[0, :]
        return carry

    jax.lax.fori_loop(0, tb // _RPB, body, 0)

    mean = mean_ref[...].reshape(tb, mean_ref.shape[2]) * (1.0 / seq_len)
    h = jnp.dot(mean, w1_ref[...],
                preferred_element_type=jnp.float32) + b1_ref[...]
    h = jnp.maximum(h, 0.0)
    logits = jnp.dot(h, w2_ref[...],
                     preferred_element_type=jnp.float32) + b2_ref[...]
    m = jnp.max(logits, axis=1, keepdims=True)
    shifted = logits - m
    lse = jnp.log(jnp.sum(jnp.exp(shifted), axis=1, keepdims=True))
    out_ref[...] = shifted - lse


def kernel(ids, emb, w1, b1, w2, b2):
    B, S = ids.shape
    V, D = emb.shape
    H = w1.shape[1]
    O = w2.shape[1]

    nb = pl.cdiv(B, _TB)
    Bp = nb * _TB
    ids_p = ids
    if Bp != B:
        ids_p = jnp.zeros((Bp, S), jnp.int32).at[:B, :].set(ids)
    ids_flat = ids_p.reshape(Bp * S)

    e3 = emb.reshape(V, 1, D)

    out = pl.pallas_call(
        functools.partial(_dan_kernel, seq_len=S, tb=_TB),
        out_shape=jax.ShapeDtypeStruct((Bp, O), jnp.float32),
        grid=(nb,),
        in_specs=[
            pl.BlockSpec(memory_space=pltpu.SMEM),
            pl.BlockSpec((V, 1, D), lambda i: (0, 0, 0)),
            pl.BlockSpec((D, H), lambda i: (0, 0)),
            pl.BlockSpec((1, H), lambda i: (0, 0)),
            pl.BlockSpec((H, O), lambda i: (0, 0)),
            pl.BlockSpec((1, O), lambda i: (0, 0)),
        ],
        out_specs=pl.BlockSpec((_TB, O), lambda i: (i, 0)),
        scratch_shapes=[pltpu.VMEM((_TB, 1, D), jnp.float32)],
        compiler_params=pltpu.CompilerParams(
            dimension_semantics=("arbitrary",)),
    )(ids_flat, e3, w1, b1, w2, b2)

    return out[:B, :]


# speedup vs baseline: 1.1240x; 1.0075x over previous
import functools

import jax
import jax.numpy as jnp
from jax.experimental import pallas as pl
from jax.experimental.pallas import tpu as pltpu

_TB = 128
_RPB = 8


def _dan_kernel(ids_smem, e3_ref, w1_ref, b1_ref, w2_ref, b2_ref,
                out_ref, mean_ref, *, seq_len, tb):
    gi = pl.program_id(0)

    def body(g, carry):
        row0 = g * _RPB
        base = (gi * tb + row0) * seq_len
        for r in range(_RPB):
            rowbase = base + r * seq_len
            acc = e3_ref[pl.ds(ids_smem[rowbase], 1), 0, :]
            for s in range(1, seq_len):
                acc = acc + e3_ref[pl.ds(ids_smem[rowbase + s], 1), 0, :]
            mean_ref[row0 + r, 0, :] = acc[0, :]
        return carry

    jax.lax.fori_loop(0, tb // _RPB, body, 0)

    mean = mean_ref[...].reshape(tb, mean_ref.shape[2]) * (1.0 / seq_len)
    h = jnp.dot(mean, w1_ref[...],
                preferred_element_type=jnp.float32) + b1_ref[...]
    h = jnp.maximum(h, 0.0)
    logits = jnp.dot(h, w2_ref[...],
                     preferred_element_type=jnp.float32) + b2_ref[...]
    m = jnp.max(logits, axis=1, keepdims=True)
    shifted = logits - m
    lse = jnp.log(jnp.sum(jnp.exp(shifted), axis=1, keepdims=True))
    out_ref[...] = shifted - lse


def kernel(ids, emb, w1, b1, w2, b2):
    B, S = ids.shape
    V, D = emb.shape
    H = w1.shape[1]
    O = w2.shape[1]

    nb = pl.cdiv(B, _TB)
    Bp = nb * _TB
    ids_p = ids
    if Bp != B:
        ids_p = jnp.zeros((Bp, S), jnp.int32).at[:B, :].set(ids)
    ids_flat = ids_p.reshape(Bp * S)

    e3 = emb.reshape(V, 1, D)

    out = pl.pallas_call(
        functools.partial(_dan_kernel, seq_len=S, tb=_TB),
        out_shape=jax.ShapeDtypeStruct((Bp, O), jnp.float32),
        grid=(nb,),
        in_specs=[
            pl.BlockSpec(memory_space=pltpu.SMEM),
            pl.BlockSpec((V, 1, D), lambda i: (0, 0, 0)),
            pl.BlockSpec((D, H), lambda i: (0, 0)),
            pl.BlockSpec((1, H), lambda i: (0, 0)),
            pl.BlockSpec((H, O), lambda i: (0, 0)),
            pl.BlockSpec((1, O), lambda i: (0, 0)),
        ],
        out_specs=pl.BlockSpec((_TB, O), lambda i: (i, 0)),
        scratch_shapes=[pltpu.VMEM((_TB, 1, D), jnp.float32)],
        compiler_params=pltpu.CompilerParams(
            dimension_semantics=("arbitrary",)),
    )(ids_flat, e3, w1, b1, w2, b2)

    return out[:B, :]


# speedup vs baseline: 1.3943x; 1.2404x over previous
import functools

import jax
import jax.numpy as jnp
from jax.experimental import pallas as pl
from jax.experimental.pallas import tpu as pltpu

_TB = 128
_RPB = 8


def _dan_kernel(ids_smem, emb_hbm, w1_ref, b1_ref, w2_ref, b2_ref,
                out_ref, e3_ref, mean_ref, sem, *, seq_len, tb):
    gi = pl.program_id(0)

    @pl.when(gi == 0)
    def _load_table():
        cp = pltpu.make_async_copy(emb_hbm, e3_ref.at[:, 0, :], sem)
        cp.start()
        cp.wait()

    def body(g, carry):
        row0 = g * _RPB
        base = (gi * tb + row0) * seq_len
        for r in range(_RPB):
            rowbase = base + r * seq_len
            acc = e3_ref[pl.ds(ids_smem[rowbase], 1), 0, :]
            for s in range(1, seq_len):
                acc = acc + e3_ref[pl.ds(ids_smem[rowbase + s], 1), 0, :]
            mean_ref[row0 + r, 0, :] = acc[0, :]
        return carry

    jax.lax.fori_loop(0, tb // _RPB, body, 0)

    mean = mean_ref[...].reshape(tb, mean_ref.shape[2]) * (1.0 / seq_len)
    h = jnp.dot(mean, w1_ref[...],
                preferred_element_type=jnp.float32) + b1_ref[...]
    h = jnp.maximum(h, 0.0)
    logits = jnp.dot(h, w2_ref[...],
                     preferred_element_type=jnp.float32) + b2_ref[...]
    m = jnp.max(logits, axis=1, keepdims=True)
    shifted = logits - m
    lse = jnp.log(jnp.sum(jnp.exp(shifted), axis=1, keepdims=True))
    out_ref[...] = shifted - lse


def kernel(ids, emb, w1, b1, w2, b2):
    B, S = ids.shape
    V, D = emb.shape
    H = w1.shape[1]
    O = w2.shape[1]

    nb = pl.cdiv(B, _TB)
    Bp = nb * _TB
    ids_p = ids
    if Bp != B:
        ids_p = jnp.zeros((Bp, S), jnp.int32).at[:B, :].set(ids)
    ids_flat = ids_p.reshape(Bp * S)

    out = pl.pallas_call(
        functools.partial(_dan_kernel, seq_len=S, tb=_TB),
        out_shape=jax.ShapeDtypeStruct((Bp, O), jnp.float32),
        grid=(nb,),
        in_specs=[
            pl.BlockSpec(memory_space=pltpu.SMEM),
            pl.BlockSpec(memory_space=pl.ANY),
            pl.BlockSpec((D, H), lambda i: (0, 0)),
            pl.BlockSpec((1, H), lambda i: (0, 0)),
            pl.BlockSpec((H, O), lambda i: (0, 0)),
            pl.BlockSpec((1, O), lambda i: (0, 0)),
        ],
        out_specs=pl.BlockSpec((_TB, O), lambda i: (i, 0)),
        scratch_shapes=[pltpu.VMEM((V, 1, D), jnp.float32),
                        pltpu.VMEM((_TB, 1, D), jnp.float32),
                        pltpu.SemaphoreType.DMA],
        compiler_params=pltpu.CompilerParams(
            dimension_semantics=("arbitrary",)),
    )(ids_flat, emb, w1, b1, w2, b2)

    return out[:B, :]


# speedup vs baseline: 1.4025x; 1.0059x over previous
import functools

import jax
import jax.numpy as jnp
from jax.experimental import pallas as pl
from jax.experimental.pallas import tpu as pltpu

_TB = 128
_RPB = 16


def _dan_kernel(ids_smem, emb_hbm, w1_ref, b1_ref, w2_ref, b2_ref,
                out_ref, e3_ref, mean_ref, sem, *, seq_len, tb):
    gi = pl.program_id(0)

    @pl.when(gi == 0)
    def _load_table():
        cp = pltpu.make_async_copy(emb_hbm, e3_ref.at[:, 0, :], sem)
        cp.start()
        cp.wait()

    def body(g, carry):
        row0 = g * _RPB
        base = (gi * tb + row0) * seq_len
        for r in range(_RPB):
            rowbase = base + r * seq_len
            acc = e3_ref[pl.ds(ids_smem[rowbase], 1), 0, :]
            for s in range(1, seq_len):
                acc = acc + e3_ref[pl.ds(ids_smem[rowbase + s], 1), 0, :]
            mean_ref[row0 + r, 0, :] = acc[0, :]
        return carry

    jax.lax.fori_loop(0, tb // _RPB, body, 0)

    mean = mean_ref[...].reshape(tb, mean_ref.shape[2]) * (1.0 / seq_len)
    h = jnp.dot(mean, w1_ref[...],
                preferred_element_type=jnp.float32) + b1_ref[...]
    h = jnp.maximum(h, 0.0)
    logits = jnp.dot(h, w2_ref[...],
                     preferred_element_type=jnp.float32) + b2_ref[...]
    m = jnp.max(logits, axis=1, keepdims=True)
    shifted = logits - m
    lse = jnp.log(jnp.sum(jnp.exp(shifted), axis=1, keepdims=True))
    out_ref[...] = shifted - lse


def kernel(ids, emb, w1, b1, w2, b2):
    B, S = ids.shape
    V, D = emb.shape
    H = w1.shape[1]
    O = w2.shape[1]

    nb = pl.cdiv(B, _TB)
    Bp = nb * _TB
    ids_p = ids
    if Bp != B:
        ids_p = jnp.zeros((Bp, S), jnp.int32).at[:B, :].set(ids)
    ids_flat = ids_p.reshape(Bp * S)

    out = pl.pallas_call(
        functools.partial(_dan_kernel, seq_len=S, tb=_TB),
        out_shape=jax.ShapeDtypeStruct((Bp, O), jnp.float32),
        grid=(nb,),
        in_specs=[
            pl.BlockSpec(memory_space=pltpu.SMEM),
            pl.BlockSpec(memory_space=pl.ANY),
            pl.BlockSpec((D, H), lambda i: (0, 0)),
            pl.BlockSpec((1, H), lambda i: (0, 0)),
            pl.BlockSpec((H, O), lambda i: (0, 0)),
            pl.BlockSpec((1, O), lambda i: (0, 0)),
        ],
        out_specs=pl.BlockSpec((_TB, O), lambda i: (i, 0)),
        scratch_shapes=[pltpu.VMEM((V, 1, D), jnp.float32),
                        pltpu.VMEM((_TB, 1, D), jnp.float32),
                        pltpu.SemaphoreType.DMA],
        compiler_params=pltpu.CompilerParams(
            dimension_semantics=("arbitrary",)),
    )(ids_flat, emb, w1, b1, w2, b2)

    return out[:B, :]


# speedup vs baseline: 1.4297x; 1.0194x over previous
import functools

import jax
import jax.numpy as jnp
from jax.experimental import pallas as pl
from jax.experimental.pallas import tpu as pltpu

_TB = 256
_RPB = 16


def _dan_kernel(ids_smem, emb_hbm, w1_ref, b1_ref, w2_ref, b2_ref,
                out_ref, e3_ref, mean_ref, sem, *, seq_len, tb):
    gi = pl.program_id(0)

    @pl.when(gi == 0)
    def _load_table():
        cp = pltpu.make_async_copy(emb_hbm, e3_ref.at[:, 0, :], sem)
        cp.start()
        cp.wait()

    def body(g, carry):
        row0 = g * _RPB
        base = (gi * tb + row0) * seq_len
        for r in range(_RPB):
            rowbase = base + r * seq_len
            acc = e3_ref[pl.ds(ids_smem[rowbase], 1), 0, :]
            for s in range(1, seq_len):
                acc = acc + e3_ref[pl.ds(ids_smem[rowbase + s], 1), 0, :]
            mean_ref[row0 + r, 0, :] = acc[0, :]
        return carry

    jax.lax.fori_loop(0, tb // _RPB, body, 0)

    mean = mean_ref[...].reshape(tb, mean_ref.shape[2]) * (1.0 / seq_len)
    h = jnp.dot(mean, w1_ref[...],
                preferred_element_type=jnp.float32) + b1_ref[...]
    h = jnp.maximum(h, 0.0)
    logits = jnp.dot(h, w2_ref[...],
                     preferred_element_type=jnp.float32) + b2_ref[...]
    m = jnp.max(logits, axis=1, keepdims=True)
    shifted = logits - m
    lse = jnp.log(jnp.sum(jnp.exp(shifted), axis=1, keepdims=True))
    out_ref[...] = shifted - lse


def kernel(ids, emb, w1, b1, w2, b2):
    B, S = ids.shape
    V, D = emb.shape
    H = w1.shape[1]
    O = w2.shape[1]

    nb = pl.cdiv(B, _TB)
    Bp = nb * _TB
    ids_p = ids
    if Bp != B:
        ids_p = jnp.zeros((Bp, S), jnp.int32).at[:B, :].set(ids)
    ids_flat = ids_p.reshape(Bp * S)

    out = pl.pallas_call(
        functools.partial(_dan_kernel, seq_len=S, tb=_TB),
        out_shape=jax.ShapeDtypeStruct((Bp, O), jnp.float32),
        grid=(nb,),
        in_specs=[
            pl.BlockSpec(memory_space=pltpu.SMEM),
            pl.BlockSpec(memory_space=pl.ANY),
            pl.BlockSpec((D, H), lambda i: (0, 0)),
            pl.BlockSpec((1, H), lambda i: (0, 0)),
            pl.BlockSpec((H, O), lambda i: (0, 0)),
            pl.BlockSpec((1, O), lambda i: (0, 0)),
        ],
        out_specs=pl.BlockSpec((_TB, O), lambda i: (i, 0)),
        scratch_shapes=[pltpu.VMEM((V, 1, D), jnp.float32),
                        pltpu.VMEM((_TB, 1, D), jnp.float32),
                        pltpu.SemaphoreType.DMA],
        compiler_params=pltpu.CompilerParams(
            dimension_semantics=("arbitrary",)),
    )(ids_flat, emb, w1, b1, w2, b2)

    return out[:B, :]


# speedup vs baseline: 1.4326x; 1.0020x over previous
import functools

import jax
import jax.numpy as jnp
from jax.experimental import pallas as pl
from jax.experimental.pallas import tpu as pltpu

_TB = 256
_RPB = 32


def _dan_kernel(ids_smem, emb_hbm, w1_ref, b1_ref, w2_ref, b2_ref,
                out_ref, e3_ref, mean_ref, sem, *, seq_len, tb):
    gi = pl.program_id(0)

    @pl.when(gi == 0)
    def _load_table():
        cp = pltpu.make_async_copy(emb_hbm, e3_ref.at[:, 0, :], sem)
        cp.start()
        cp.wait()

    def body(g, carry):
        row0 = g * _RPB
        base = (gi * tb + row0) * seq_len
        for r in range(_RPB):
            rowbase = base + r * seq_len
            acc = e3_ref[pl.ds(ids_smem[rowbase], 1), 0, :]
            for s in range(1, seq_len):
                acc = acc + e3_ref[pl.ds(ids_smem[rowbase + s], 1), 0, :]
            mean_ref[row0 + r, 0, :] = acc[0, :]
        return carry

    jax.lax.fori_loop(0, tb // _RPB, body, 0)

    mean = mean_ref[...].reshape(tb, mean_ref.shape[2]) * (1.0 / seq_len)
    h = jnp.dot(mean, w1_ref[...],
                preferred_element_type=jnp.float32) + b1_ref[...]
    h = jnp.maximum(h, 0.0)
    logits = jnp.dot(h, w2_ref[...],
                     preferred_element_type=jnp.float32) + b2_ref[...]
    m = jnp.max(logits, axis=1, keepdims=True)
    shifted = logits - m
    lse = jnp.log(jnp.sum(jnp.exp(shifted), axis=1, keepdims=True))
    out_ref[...] = shifted - lse


def kernel(ids, emb, w1, b1, w2, b2):
    B, S = ids.shape
    V, D = emb.shape
    H = w1.shape[1]
    O = w2.shape[1]

    nb = pl.cdiv(B, _TB)
    Bp = nb * _TB
    ids_p = ids
    if Bp != B:
        ids_p = jnp.zeros((Bp, S), jnp.int32).at[:B, :].set(ids)
    ids_flat = ids_p.reshape(Bp * S)

    out = pl.pallas_call(
        functools.partial(_dan_kernel, seq_len=S, tb=_TB),
        out_shape=jax.ShapeDtypeStruct((Bp, O), jnp.float32),
        grid=(nb,),
        in_specs=[
            pl.BlockSpec(memory_space=pltpu.SMEM),
            pl.BlockSpec(memory_space=pl.ANY),
            pl.BlockSpec((D, H), lambda i: (0, 0)),
            pl.BlockSpec((1, H), lambda i: (0, 0)),
            pl.BlockSpec((H, O), lambda i: (0, 0)),
            pl.BlockSpec((1, O), lambda i: (0, 0)),
        ],
        out_specs=pl.BlockSpec((_TB, O), lambda i: (i, 0)),
        scratch_shapes=[pltpu.VMEM((V, 1, D), jnp.float32),
                        pltpu.VMEM((_TB, 1, D), jnp.float32),
                        pltpu.SemaphoreType.DMA],
        compiler_params=pltpu.CompilerParams(
            dimension_semantics=("arbitrary",)),
    )(ids_flat, emb, w1, b1, w2, b2)

    return out[:B, :]


# speedup vs baseline: 1.4329x; 1.0003x over previous
import functools

import jax
import jax.numpy as jnp
from jax.experimental import pallas as pl
from jax.experimental.pallas import tpu as pltpu

_TB = 256
_RPB = 64


def _dan_kernel(ids_smem, emb_hbm, w1_ref, b1_ref, w2_ref, b2_ref,
                out_ref, e3_ref, mean_ref, sem, *, seq_len, tb):
    gi = pl.program_id(0)

    @pl.when(gi == 0)
    def _load_table():
        cp = pltpu.make_async_copy(emb_hbm, e3_ref.at[:, 0, :], sem)
        cp.start()
        cp.wait()

    def body(g, carry):
        row0 = g * _RPB
        base = (gi * tb + row0) * seq_len
        for r in range(_RPB):
            rowbase = base + r * seq_len
            acc = e3_ref[pl.ds(ids_smem[rowbase], 1), 0, :]
            for s in range(1, seq_len):
                acc = acc + e3_ref[pl.ds(ids_smem[rowbase + s], 1), 0, :]
            mean_ref[row0 + r, 0, :] = acc[0, :]
        return carry

    jax.lax.fori_loop(0, tb // _RPB, body, 0)

    mean = mean_ref[...].reshape(tb, mean_ref.shape[2]) * (1.0 / seq_len)
    h = jnp.dot(mean, w1_ref[...],
                preferred_element_type=jnp.float32) + b1_ref[...]
    h = jnp.maximum(h, 0.0)
    logits = jnp.dot(h, w2_ref[...],
                     preferred_element_type=jnp.float32) + b2_ref[...]
    m = jnp.max(logits, axis=1, keepdims=True)
    shifted = logits - m
    lse = jnp.log(jnp.sum(jnp.exp(shifted), axis=1, keepdims=True))
    out_ref[...] = shifted - lse


def kernel(ids, emb, w1, b1, w2, b2):
    B, S = ids.shape
    V, D = emb.shape
    H = w1.shape[1]
    O = w2.shape[1]

    nb = pl.cdiv(B, _TB)
    Bp = nb * _TB
    ids_p = ids
    if Bp != B:
        ids_p = jnp.zeros((Bp, S), jnp.int32).at[:B, :].set(ids)
    ids_flat = ids_p.reshape(Bp * S)

    out = pl.pallas_call(
        functools.partial(_dan_kernel, seq_len=S, tb=_TB),
        out_shape=jax.ShapeDtypeStruct((Bp, O), jnp.float32),
        grid=(nb,),
        in_specs=[
            pl.BlockSpec(memory_space=pltpu.SMEM),
            pl.BlockSpec(memory_space=pl.ANY),
            pl.BlockSpec((D, H), lambda i: (0, 0)),
            pl.BlockSpec((1, H), lambda i: (0, 0)),
            pl.BlockSpec((H, O), lambda i: (0, 0)),
            pl.BlockSpec((1, O), lambda i: (0, 0)),
        ],
        out_specs=pl.BlockSpec((_TB, O), lambda i: (i, 0)),
        scratch_shapes=[pltpu.VMEM((V, 1, D), jnp.float32),
                        pltpu.VMEM((_TB, 1, D), jnp.float32),
                        pltpu.SemaphoreType.DMA],
        compiler_params=pltpu.CompilerParams(
            dimension_semantics=("arbitrary",)),
    )(ids_flat, emb, w1, b1, w2, b2)

    return out[:B, :]


# speedup vs baseline: 1.4350x; 1.0014x over previous
import functools

import jax
import jax.numpy as jnp
from jax.experimental import pallas as pl
from jax.experimental.pallas import tpu as pltpu

_TB = 256
_RPB = 32


def _dan_kernel(ids_smem, emb_hbm, w1_ref, b1_ref, w2_ref, b2_ref,
                out_ref, e3_ref, mean_ref, sem, *, seq_len, tb):
    gi = pl.program_id(0)

    @pl.when(gi == 0)
    def _load_table():
        cp = pltpu.make_async_copy(emb_hbm, e3_ref.at[:, 0, :], sem)
        cp.start()
        cp.wait()

    nw = (seq_len + 1) // 2
    odd = seq_len % 2

    def body(g, carry):
        row0 = g * _RPB
        base = (gi * tb + row0) * nw
        for r in range(_RPB):
            rowbase = base + r * nw
            w0 = ids_smem[rowbase]
            acc = e3_ref[pl.ds(w0 & 0xFFFF, 1), 0, :]
            if seq_len > 1:
                acc = acc + e3_ref[pl.ds(w0 >> 16, 1), 0, :]
            for sp in range(1, nw):
                w = ids_smem[rowbase + sp]
                acc = acc + e3_ref[pl.ds(w & 0xFFFF, 1), 0, :]
                if sp < nw - 1 or not odd:
                    acc = acc + e3_ref[pl.ds(w >> 16, 1), 0, :]
            mean_ref[row0 + r, 0, :] = acc[0, :]
        return carry

    jax.lax.fori_loop(0, tb // _RPB, body, 0)

    mean = mean_ref[...].reshape(tb, mean_ref.shape[2]) * (1.0 / seq_len)
    h = jnp.dot(mean, w1_ref[...],
                preferred_element_type=jnp.float32) + b1_ref[...]
    h = jnp.maximum(h, 0.0)
    logits = jnp.dot(h, w2_ref[...],
                     preferred_element_type=jnp.float32) + b2_ref[...]
    m = jnp.max(logits, axis=1, keepdims=True)
    shifted = logits - m
    lse = jnp.log(jnp.sum(jnp.exp(shifted), axis=1, keepdims=True))
    out_ref[...] = shifted - lse


def kernel(ids, emb, w1, b1, w2, b2):
    B, S = ids.shape
    V, D = emb.shape
    H = w1.shape[1]
    O = w2.shape[1]

    nb = pl.cdiv(B, _TB)
    Bp = nb * _TB
    ids_p = ids
    if Bp != B:
        ids_p = jnp.zeros((Bp, S), jnp.int32).at[:B, :].set(ids)
    if S % 2:
        ids_p = jnp.concatenate(
            [ids_p, jnp.zeros((Bp, 1), jnp.int32)], axis=1)
    ids_pack = (ids_p[:, 0::2] | (ids_p[:, 1::2] << 16)).reshape(-1)

    out = pl.pallas_call(
        functools.partial(_dan_kernel, seq_len=S, tb=_TB),
        out_shape=jax.ShapeDtypeStruct((Bp, O), jnp.float32),
        grid=(nb,),
        in_specs=[
            pl.BlockSpec(memory_space=pltpu.SMEM),
            pl.BlockSpec(memory_space=pl.ANY),
            pl.BlockSpec((D, H), lambda i: (0, 0)),
            pl.BlockSpec((1, H), lambda i: (0, 0)),
            pl.BlockSpec((H, O), lambda i: (0, 0)),
            pl.BlockSpec((1, O), lambda i: (0, 0)),
        ],
        out_specs=pl.BlockSpec((_TB, O), lambda i: (i, 0)),
        scratch_shapes=[pltpu.VMEM((V, 1, D), jnp.float32),
                        pltpu.VMEM((_TB, 1, D), jnp.float32),
                        pltpu.SemaphoreType.DMA],
        compiler_params=pltpu.CompilerParams(
            dimension_semantics=("arbitrary",)),
    )(ids_pack, emb, w1, b1, w2, b2)

    return out[:B, :]


# speedup vs baseline: 1.4475x; 1.0087x over previous
import functools

import jax
import jax.numpy as jnp
from jax.experimental import pallas as pl
from jax.experimental.pallas import tpu as pltpu

_TB = 256
_RPB = 32


def _dan_kernel(ids_smem, emb_hbm, w1_ref, b1_ref, w2_ref, b2_ref,
                out_ref, e3_ref, mean_ref, sem, *, seq_len, tb):
    gi = pl.program_id(0)

    @pl.when(gi == 0)
    def _load_table():
        v = e3_ref.shape[0]
        vq = v // 4
        cps = [pltpu.make_async_copy(
                   emb_hbm.at[pl.ds(k * vq, vq), :],
                   e3_ref.at[pl.ds(k * vq, vq), 0, :],
                   sem.at[k])
               for k in range(4)]
        for cp in cps:
            cp.start()
        for cp in cps:
            cp.wait()

    nw = (seq_len + 1) // 2
    odd = seq_len % 2

    def body(g, carry):
        row0 = g * _RPB
        base = (gi * tb + row0) * nw
        for r in range(_RPB):
            rowbase = base + r * nw
            w0 = ids_smem[rowbase]
            acc = e3_ref[pl.ds(w0 & 0xFFFF, 1), 0, :]
            if seq_len > 1:
                acc = acc + e3_ref[pl.ds(w0 >> 16, 1), 0, :]
            for sp in range(1, nw):
                w = ids_smem[rowbase + sp]
                acc = acc + e3_ref[pl.ds(w & 0xFFFF, 1), 0, :]
                if sp < nw - 1 or not odd:
                    acc = acc + e3_ref[pl.ds(w >> 16, 1), 0, :]
            mean_ref[row0 + r, 0, :] = acc[0, :]
        return carry

    jax.lax.fori_loop(0, tb // _RPB, body, 0)

    mean = mean_ref[...].reshape(tb, mean_ref.shape[2]) * (1.0 / seq_len)
    h = jnp.dot(mean, w1_ref[...],
                preferred_element_type=jnp.float32) + b1_ref[...]
    h = jnp.maximum(h, 0.0)
    logits = jnp.dot(h, w2_ref[...],
                     preferred_element_type=jnp.float32) + b2_ref[...]
    m = jnp.max(logits, axis=1, keepdims=True)
    shifted = logits - m
    lse = jnp.log(jnp.sum(jnp.exp(shifted), axis=1, keepdims=True))
    out_ref[...] = shifted - lse


def kernel(ids, emb, w1, b1, w2, b2):
    B, S = ids.shape
    V, D = emb.shape
    H = w1.shape[1]
    O = w2.shape[1]

    nb = pl.cdiv(B, _TB)
    Bp = nb * _TB
    ids_p = ids
    if Bp != B:
        ids_p = jnp.zeros((Bp, S), jnp.int32).at[:B, :].set(ids)
    if S % 2:
        ids_p = jnp.concatenate(
            [ids_p, jnp.zeros((Bp, 1), jnp.int32)], axis=1)
    ids_pack = (ids_p[:, 0::2] | (ids_p[:, 1::2] << 16)).reshape(-1)

    out = pl.pallas_call(
        functools.partial(_dan_kernel, seq_len=S, tb=_TB),
        out_shape=jax.ShapeDtypeStruct((Bp, O), jnp.float32),
        grid=(nb,),
        in_specs=[
            pl.BlockSpec(memory_space=pltpu.SMEM),
            pl.BlockSpec(memory_space=pl.ANY),
            pl.BlockSpec((D, H), lambda i: (0, 0)),
            pl.BlockSpec((1, H), lambda i: (0, 0)),
            pl.BlockSpec((H, O), lambda i: (0, 0)),
            pl.BlockSpec((1, O), lambda i: (0, 0)),
        ],
        out_specs=pl.BlockSpec((_TB, O), lambda i: (i, 0)),
        scratch_shapes=[pltpu.VMEM((V, 1, D), jnp.float32),
                        pltpu.VMEM((_TB, 1, D), jnp.float32),
                        pltpu.SemaphoreType.DMA((4,))],
        compiler_params=pltpu.CompilerParams(
            dimension_semantics=("arbitrary",)),
    )(ids_pack, emb, w1, b1, w2, b2)

    return out[:B, :]
